# Initial kernel scaffold; baseline (speedup 1.0000x reference)
import jax
import jax.numpy as jnp
from jax.experimental import pallas as pl


def kernel(ent_table, rel_table, W, b, node_idx, edge_index, edge_type, batch_idx):
    raise NotImplementedError("write your pallas kernel here")


# speedup vs baseline: 3.5258x; 3.5258x over previous
import functools

import jax
import jax.numpy as jnp
from jax import lax
from jax.experimental import pallas as pl
from jax.experimental.pallas import tpu as pltpu
from jax.experimental.pallas import tpu_sc as plsc

N = 10000
NP = 10240
E = 320000
D = 128
H = 4
R = 64
B = 1024

_NC = 2
_NS = 16
_NW = _NC * _NS
_C = 128

_mesh = plsc.VectorSubcoreMesh(core_axis_name="c", subcore_axis_name="s")


def _wid():
    return lax.axis_index("s") * _NC + lax.axis_index("c")


@functools.partial(
    pl.kernel,
    out_type=jax.ShapeDtypeStruct((NP, D), jnp.float32),
    mesh=_mesh,
    scratch_types=[
        pltpu.VMEM((_C,), jnp.int32),
        pltpu.VMEM((_C, D), jnp.float32),
        pltpu.SemaphoreType.DMA,
    ],
)
def _sc_embed(table_hbm, idx_hbm, out_hbm, idx_v, rows_v, sem):
    w = _wid()
    nch_total = NP // _C
    base_n = nch_total // _NW
    rem = nch_total - base_n * _NW
    nch = base_n + jnp.where(w < rem, 1, 0)

    def body(j, carry):
        cid = w + j * _NW
        r0 = cid * _C
        pltpu.sync_copy(idx_hbm.at[pl.ds(r0, _C)], idx_v)
        pltpu.async_copy(table_hbm.at[idx_v], rows_v, sem).wait()
        pltpu.sync_copy(rows_v, out_hbm.at[pl.ds(r0, _C)])
        return carry

    lax.fori_loop(0, nch, body, 0)


@functools.partial(
    pl.kernel,
    out_type=jax.ShapeDtypeStruct((_NC, NP, D), jnp.float32),
    mesh=_mesh,
    scratch_types=[
        pltpu.VMEM((_C,), jnp.int32),
        pltpu.VMEM((_C,), jnp.int32),
        pltpu.VMEM((_C,), jnp.int32),
        pltpu.VMEM((_C, D), jnp.float32),
        pltpu.VMEM((_C, D), jnp.float32),
        pltpu.VMEM_SHARED((NP, D), jnp.float32),
        pltpu.SemaphoreType.DMA,
        pltpu.SemaphoreType.DMA,
    ],
)
def _sc_edge_pass(h_hbm, src_hbm, dst_hbm, typ_hbm, rel_hbm, zeros_hbm,
                  out_hbm, sidx, didx, tidx, hbuf, rbuf, acc, sem1, sem2):
    c = lax.axis_index("c")
    s = lax.axis_index("s")
    w = _wid()

    pltpu.sync_copy(zeros_hbm, hbuf)
    for j in range(5):
        pltpu.sync_copy(hbuf, acc.at[pl.ds(s * 640 + j * _C, _C)])
    plsc.subcore_barrier()

    nch_total = E // _C
    base_n = nch_total // _NW
    rem = nch_total - base_n * _NW
    nch = base_n + jnp.where(w < rem, 1, 0)

    def body(j, carry):
        cid = w + j * _NW
        e0 = cid * _C
        pltpu.sync_copy(src_hbm.at[pl.ds(e0, _C)], sidx)
        pltpu.sync_copy(typ_hbm.at[pl.ds(e0, _C)], tidx)
        pltpu.sync_copy(dst_hbm.at[pl.ds(e0, _C)], didx)
        d1 = pltpu.async_copy(h_hbm.at[sidx], hbuf, sem1)
        d2 = pltpu.async_copy(rel_hbm.at[tidx], rbuf, sem2)
        d1.wait()
        d2.wait()

        def mul_row(i, carry2):
            for k in range(D // 16):
                sl = pl.ds(k * 16, 16)
                hbuf[i, sl] = hbuf[i, sl] * rbuf[i, sl]
            return carry2

        lax.fori_loop(0, _C, mul_row, 0)
        pltpu.sync_copy(hbuf, acc.at[didx], add=True)
        return carry

    lax.fori_loop(0, nch, body, 0)
    plsc.subcore_barrier()

    pltpu.sync_copy(acc.at[pl.ds(s * 640, 640)], out_hbm.at[c, pl.ds(s * 640, 640)])


@functools.partial(
    pl.kernel,
    out_type=jax.ShapeDtypeStruct((_NC, H * B, D), jnp.float32),
    mesh=_mesh,
    scratch_types=[
        pltpu.VMEM((_C,), jnp.int32),
        pltpu.VMEM((_C, D), jnp.float32),
        pltpu.VMEM_SHARED((H * B, D), jnp.float32),
        pltpu.SemaphoreType.DMA,
    ],
)
def _sc_readout(h_hbm, idx_hbm, zeros_hbm, out_hbm, idx_v, rows_v, acc, sem):
    c = lax.axis_index("c")
    s = lax.axis_index("s")
    w = _wid()

    pltpu.sync_copy(zeros_hbm, rows_v)
    for j in range(2):
        pltpu.sync_copy(rows_v, acc.at[pl.ds(s * 256 + j * _C, _C)])
    plsc.subcore_barrier()

    nch = (H * NP) // _C // _NW

    def body(j, carry):
        cid = w + j * _NW
        r0 = cid * _C
        pltpu.sync_copy(idx_hbm.at[pl.ds(r0, _C)], idx_v)
        pltpu.sync_copy(h_hbm.at[pl.ds(r0, _C)], rows_v)
        pltpu.sync_copy(rows_v, acc.at[idx_v], add=True)
        return carry

    lax.fori_loop(0, nch, body, 0)
    plsc.subcore_barrier()
    pltpu.sync_copy(acc.at[pl.ds(s * 256, 256)], out_hbm.at[c, pl.ds(s * 256, 256)])


_BLK = 1280


def _tc_linear_body(p_ref, w_ref, b_ref, o_ref):
    acc = p_ref[0, 0] + p_ref[0, 1]
    y = jnp.dot(acc, w_ref[0], preferred_element_type=jnp.float32)
    y = jnp.maximum(y + b_ref[0], 0.0)
    nb = pl.program_id(1)
    rows = nb * _BLK + lax.broadcasted_iota(jnp.int32, (_BLK, D), 0)
    o_ref[0] = jnp.where(rows < N, y, 0.0)


def _tc_linear(partials, Wl, bl, shared_partials):
    if shared_partials:
        p_map = lambda hp, nb: (0, 0, nb, 0)
    else:
        p_map = lambda hp, nb: (hp, 0, nb, 0)
    return pl.pallas_call(
        _tc_linear_body,
        grid=(H, NP // _BLK),
        in_specs=[
            pl.BlockSpec((1, 2, _BLK, D), p_map),
            pl.BlockSpec((1, D, D), lambda hp, nb: (hp, 0, 0)),
            pl.BlockSpec((1, 1, D), lambda hp, nb: (hp, 0, 0)),
        ],
        out_specs=pl.BlockSpec((1, _BLK, D), lambda hp, nb: (hp, nb, 0)),
        out_shape=jax.ShapeDtypeStruct((H, NP, D), jnp.float32),
    )(partials, Wl, bl)


def _tc_combine_body(p_ref, o_ref):
    o_ref[...] = p_ref[0, 0] + p_ref[1, 0]


def _tc_combine(ro):
    return pl.pallas_call(
        _tc_combine_body,
        grid=(H,),
        in_specs=[pl.BlockSpec((2, 1, B, D), lambda hp: (0, hp, 0, 0))],
        out_specs=pl.BlockSpec((B, D), lambda hp: (0, hp)),
        out_shape=jax.ShapeDtypeStruct((B, H * D), jnp.float32),
    )(ro)


def kernel(ent_table, rel_table, W, b, node_idx, edge_index, edge_type, batch_idx):
    f32 = jnp.float32
    node_idx_p = jnp.pad(node_idx.astype(jnp.int32), (0, NP - N))
    src = edge_index[0].astype(jnp.int32)
    dst = edge_index[1].astype(jnp.int32)
    etyp = edge_type.astype(jnp.int32)
    zeros128 = jnp.zeros((_C, D), f32)

    bidx_p = jnp.pad(batch_idx.astype(jnp.int32), (0, NP - N))
    idx4 = (bidx_p[None, :] + (jnp.arange(H, dtype=jnp.int32) * B)[:, None])
    idx4 = idx4.reshape(H * NP)

    ent_table = ent_table.astype(f32)
    rel_table = rel_table.astype(f32)
    W = W.astype(f32)
    b = b.astype(f32)

    x = _sc_embed(ent_table, node_idx_p)

    p0 = _sc_edge_pass(x, src, dst, etyp, rel_table, zeros128)
    h1 = _tc_linear(p0[None], W[:, 0], b[:, 0, None, :], True)

    p1 = jnp.stack([
        _sc_edge_pass(h1[hp], src, dst, etyp, rel_table, zeros128)
        for hp in range(H)
    ])
    h2 = _tc_linear(p1, W[:, 1], b[:, 1, None, :], False)

    ro = _sc_readout(h2.reshape(H * NP, D), idx4, zeros128)
    out = _tc_combine(ro.reshape(_NC, H, B, D))
    return out
